# Initial kernel scaffold; baseline (speedup 1.0000x reference)
import jax
import jax.numpy as jnp
from jax.experimental import pallas as pl


def kernel(x, W1, W2, W3, W4, W5):
    raise NotImplementedError("write your pallas kernel here")


# speedup vs baseline: 5.2658x; 5.2658x over previous
import functools
import jax
import jax.numpy as jnp
from jax import lax
from jax.experimental import pallas as pl
from jax.experimental.pallas import tpu as pltpu
from jax.experimental.pallas import tpu_sc as plsc

EPSN = 1e-5
K = 20
N = 1024
NB = 4

try:
    _sc_info = plsc.get_sparse_core_info()
    NC, NS = _sc_info.num_cores, _sc_info.num_subcores
except Exception:
    NC, NS = 2, 16
NW = NC * NS


def _lrelu(x):
    return jnp.where(x >= 0, x, 0.2 * x)


def _pd_topk_body(x_ref, idx_ref):
    b = pl.program_id(0)
    x = x_ref[0]
    g = lax.dot_general(x, x, (((0,), (0,)), ((), ())),
                        preferred_element_type=jnp.float32)
    s = jnp.sum(x * x, axis=0)
    pd = (2.0 * g - s[None, :]) - s[:, None]

    n = pd.shape[0]
    col = lax.broadcasted_iota(jnp.int32, (n, n), 1)
    work = pd
    neg = jnp.float32(-jnp.inf)
    for j in range(K):
        m = jnp.max(work, axis=1, keepdims=True)
        cand = jnp.where(work == m, col, n)
        sel = jnp.min(cand, axis=1, keepdims=True)
        idx_ref[0, :, j] = sel[:, 0] + b * n
        work = jnp.where(col == sel, neg, work)


def _pd_topk(x_cn):
    B, Cp, n = x_cn.shape
    return pl.pallas_call(
        _pd_topk_body,
        grid=(B,),
        in_specs=[pl.BlockSpec((1, Cp, n), lambda b: (b, 0, 0))],
        out_specs=pl.BlockSpec((1, n, K), lambda b: (b, 0, 0)),
        out_shape=jax.ShapeDtypeStruct((B, n, K), jnp.int32),
    )(x_cn)


def _gather_diff(xr, idx1d, Cp):
    P = 8
    RPC = P * K
    NCH = (NB * N) // (NW * P)
    TC16 = Cp // 16

    mesh = plsc.VectorSubcoreMesh(core_axis_name="c", subcore_axis_name="s")

    @functools.partial(
        pl.kernel,
        mesh=mesh,
        compiler_params=pltpu.CompilerParams(use_tc_tiling_on_sc=False),
        out_type=jax.ShapeDtypeStruct((NB * N * K, 2 * Cp), jnp.float32),
        scratch_types=[
            pltpu.VMEM((RPC,), jnp.int32),
            pltpu.VMEM((RPC, Cp), jnp.float32),
            pltpu.VMEM((P, Cp), jnp.float32),
            pltpu.VMEM((RPC, 2 * Cp), jnp.float32),
            pltpu.SemaphoreType.DMA,
            pltpu.SemaphoreType.DMA,
        ],
    )
    def sck(xr_hbm, idx_hbm, f_hbm, idx_v, rows_v, xc_v, out_v, sem0, sem1):
        wid = lax.axis_index("s") * NC + lax.axis_index("c")

        def chunk_body(ch, carry):
            pt0 = wid * (NCH * P) + ch * P
            r0 = pt0 * K
            pltpu.sync_copy(idx_hbm.at[pl.ds(r0, RPC)], idx_v)
            cp0 = pltpu.async_copy(
                xr_hbm.at[idx_v.at[pl.ds(0, 80)]], rows_v.at[pl.ds(0, 80)],
                sem0)
            cp1 = pltpu.async_copy(
                xr_hbm.at[idx_v.at[pl.ds(80, 80)]], rows_v.at[pl.ds(80, 80)],
                sem1)
            pltpu.sync_copy(xr_hbm.at[pl.ds(pt0, P)], xc_v)
            cp0.wait()
            cp1.wait()

            def p_body(p, c2):
                def j_body(j, c3):
                    r = p * K + j
                    for t in range(TC16):
                        c = xc_v[p, pl.ds(16 * t, 16)]
                        d = rows_v[r, pl.ds(16 * t, 16)] - c
                        out_v[r, pl.ds(16 * t, 16)] = d
                        out_v[r, pl.ds(Cp + 16 * t, 16)] = c
                    return c3
                return lax.fori_loop(0, K, j_body, c2)

            lax.fori_loop(0, P, p_body, 0)
            pltpu.sync_copy(out_v, f_hbm.at[pl.ds(r0, RPC)])
            return carry

        lax.fori_loop(0, NCH, chunk_body, 0)

    return sck(xr, idx1d)


def _conv_faithful_body(f_ref, w_ref, y4_ref, gmax_ref):
    fb = f_ref[0]
    chn = fb.shape[0]
    f2 = fb.reshape(chn * K, fb.shape[2])
    y = lax.dot_general(f2, w_ref[...], (((1,), (1,)), ((), ())),
                        preferred_element_type=jnp.float32)
    o = y.shape[1]
    y3 = y.reshape(chn, K, o)
    y4_ref[0] = y3
    gmax = y3[:, 0, :]
    for j in range(1, K):
        gmax = jnp.maximum(gmax, y3[:, j, :])
    gmax_ref[0] = gmax


def _conv_fast_body(f_ref, w_ref, gmax_ref, s1_ref, s2_ref):
    i = pl.program_id(1)
    fb = f_ref[0]
    chn = fb.shape[0]
    f2 = fb.reshape(chn * K, fb.shape[2])
    y = lax.dot_general(f2, w_ref[...], (((1,), (1,)), ((), ())),
                        preferred_element_type=jnp.float32)
    o = y.shape[1]
    y3 = y.reshape(chn, K, o)
    gmax = y3[:, 0, :]
    for j in range(1, K):
        gmax = jnp.maximum(gmax, y3[:, j, :])
    gmax_ref[0] = gmax
    s1 = jnp.sum(y, axis=0)[None, :]
    s2 = jnp.sum(y * y, axis=0)[None, :]

    @pl.when(i == 0)
    def _():
        s1_ref[0] = s1
        s2_ref[0] = s2

    @pl.when(i > 0)
    def _():
        s1_ref[0] += s1
        s2_ref[0] += s2


def _conv_faithful(f4, wp):
    B, n, k, c2 = f4.shape
    O = wp.shape[0]
    CHN = 512
    return pl.pallas_call(
        _conv_faithful_body,
        grid=(B, n // CHN),
        in_specs=[
            pl.BlockSpec((1, CHN, k, c2), lambda b, i: (b, i, 0, 0)),
            pl.BlockSpec((O, c2), lambda b, i: (0, 0)),
        ],
        out_specs=(
            pl.BlockSpec((1, CHN, k, O), lambda b, i: (b, i, 0, 0)),
            pl.BlockSpec((1, CHN, O), lambda b, i: (b, i, 0)),
        ),
        out_shape=(
            jax.ShapeDtypeStruct((B, n, k, O), jnp.float32),
            jax.ShapeDtypeStruct((B, n, O), jnp.float32),
        ),
    )(f4, wp)


def _conv_fast(f4, wp):
    B, n, k, c2 = f4.shape
    O = wp.shape[0]
    CHN = 512
    return pl.pallas_call(
        _conv_fast_body,
        grid=(B, n // CHN),
        in_specs=[
            pl.BlockSpec((1, CHN, k, c2), lambda b, i: (b, i, 0, 0)),
            pl.BlockSpec((O, c2), lambda b, i: (0, 0)),
        ],
        out_specs=(
            pl.BlockSpec((1, CHN, O), lambda b, i: (b, i, 0)),
            pl.BlockSpec((1, 1, O), lambda b, i: (b, 0, 0)),
            pl.BlockSpec((1, 1, O), lambda b, i: (b, 0, 0)),
        ),
        out_shape=(
            jax.ShapeDtypeStruct((B, n, O), jnp.float32),
            jax.ShapeDtypeStruct((B, 1, O), jnp.float32),
            jax.ShapeDtypeStruct((B, 1, O), jnp.float32),
        ),
    )(f4, wp)


def _finalize_mv_body(gmax_ref, m_ref, v_ref, xn_ref):
    scale = jnp.sqrt(v_ref[0] + EPSN)
    xn_ref[0] = _lrelu((gmax_ref[0] - m_ref[0]) / scale)


def _finalize_mv(gmax, m, v):
    B, n, O = gmax.shape
    return pl.pallas_call(
        _finalize_mv_body,
        grid=(B,),
        in_specs=[
            pl.BlockSpec((1, n, O), lambda b: (b, 0, 0)),
            pl.BlockSpec((1, 1, O), lambda b: (b, 0, 0)),
            pl.BlockSpec((1, 1, O), lambda b: (b, 0, 0)),
        ],
        out_specs=pl.BlockSpec((1, n, O), lambda b: (b, 0, 0)),
        out_shape=jax.ShapeDtypeStruct((B, n, O), jnp.float32),
    )(gmax, m, v)


def _finalize_fast_body(gmax_ref, s1_ref, s2_ref, xn_ref):
    nk = jnp.float32(N * K)
    m = s1_ref[0] / nk
    v = s2_ref[0] / nk - m * m
    scale = jnp.sqrt(v + EPSN)
    xn_ref[0] = _lrelu((gmax_ref[0] - m) / scale)


def _finalize_fast(gmax, s1, s2):
    B, n, O = gmax.shape
    return pl.pallas_call(
        _finalize_fast_body,
        grid=(B,),
        in_specs=[
            pl.BlockSpec((1, n, O), lambda b: (b, 0, 0)),
            pl.BlockSpec((1, 1, O), lambda b: (b, 0, 0)),
            pl.BlockSpec((1, 1, O), lambda b: (b, 0, 0)),
        ],
        out_specs=pl.BlockSpec((1, n, O), lambda b: (b, 0, 0)),
        out_shape=jax.ShapeDtypeStruct((B, n, O), jnp.float32),
    )(gmax, s1, s2)


def _final_body(x1_ref, x2_ref, x3_ref, x4_ref, w5_ref, out_ref):
    cat = jnp.concatenate(
        [x1_ref[0], x2_ref[0], x3_ref[0], x4_ref[0]], axis=1)
    y = lax.dot_general(w5_ref[...], cat, (((1,), (1,)), ((), ())),
                        preferred_element_type=jnp.float32)
    n = jnp.float32(y.shape[1])
    m = jnp.sum(y, axis=1, keepdims=True) / n
    v = jnp.sum(y * y, axis=1, keepdims=True) / n - m * m
    out_ref[0] = _lrelu((y - m) / jnp.sqrt(v + EPSN))


def _final(x1, x2, x3, x4, w5):
    B, n, _ = x1.shape
    return pl.pallas_call(
        _final_body,
        grid=(B,),
        in_specs=[
            pl.BlockSpec((1, n, x1.shape[2]), lambda b: (b, 0, 0)),
            pl.BlockSpec((1, n, x2.shape[2]), lambda b: (b, 0, 0)),
            pl.BlockSpec((1, n, x3.shape[2]), lambda b: (b, 0, 0)),
            pl.BlockSpec((1, n, x4.shape[2]), lambda b: (b, 0, 0)),
            pl.BlockSpec((512, 512), lambda b: (0, 0)),
        ],
        out_specs=pl.BlockSpec((1, 512, n), lambda b: (b, 0, 0)),
        out_shape=jax.ShapeDtypeStruct((B, 512, n), jnp.float32),
    )(x1, x2, x3, x4, w5)


def _pad_w(W, C, Cp):
    O = W.shape[0]
    wp = jnp.zeros((O, 2 * Cp), jnp.float32)
    return wp.at[:, :C].set(W[:, :C]).at[:, Cp:Cp + C].set(W[:, C:])


def _block_faithful(x_cn, xr, W, C):
    Cp = xr.shape[1]
    idx = _pd_topk(x_cn)
    f = _gather_diff(xr, idx.reshape(-1), Cp)
    f4 = f.reshape(NB, N, K, 2 * Cp)
    y4, gmax = _conv_faithful(f4, _pad_w(W, C, Cp))
    m = jnp.mean(y4, axis=(1, 2))[:, None, :]
    v = jnp.var(y4, axis=(1, 2))[:, None, :]
    return _finalize_mv(gmax, m, v)


def _block_fast(x_cn, xr, W, C):
    Cp = xr.shape[1]
    idx = _pd_topk(x_cn)
    f = _gather_diff(xr, idx.reshape(-1), Cp)
    f4 = f.reshape(NB, N, K, 2 * Cp)
    gmax, s1, s2 = _conv_fast(f4, _pad_w(W, C, Cp))
    return _finalize_fast(gmax, s1, s2)


def kernel(x, W1, W2, W3, W4, W5):
    B = x.shape[0]
    x_cn1 = jnp.pad(jnp.transpose(x, (0, 2, 1)), ((0, 0), (0, 5), (0, 0)))
    xr1 = jnp.pad(x.reshape(B * N, 3), ((0, 0), (0, 13)))
    x1 = _block_faithful(x_cn1, xr1, W1, 3)

    x2 = _block_faithful(jnp.transpose(x1, (0, 2, 1)),
                         x1.reshape(B * N, 64), W2, 64)
    x3 = _block_faithful(jnp.transpose(x2, (0, 2, 1)),
                         x2.reshape(B * N, 64), W3, 64)
    x4 = _block_fast(jnp.transpose(x3, (0, 2, 1)),
                     x3.reshape(B * N, 128), W4, 128)

    return _final(x1, x2, x3, x4, W5)
